# baseline (speedup 1.0000x reference)
import functools

import jax
import jax.numpy as jnp
from jax import lax
from jax.experimental import pallas as pl
from jax.experimental.pallas import tpu as pltpu
from jax.experimental.pallas import tpu_sc as plsc

N = 10000
NE = 320000
DIN, DH, DOUT = 128, 128, 64
DHH = DH // 2
NC, NS = 2, 16
NW = NC * NS
CH = 128
NCHW = 80
NCHS = 160
EROWS = NW * NCHW
NEP = EROWS * CH
NROWS = 10112
STRIPE = NROWS // NS
DEGW = 8
RB = 1000


def _sc_mesh():
    return plsc.VectorSubcoreMesh(core_axis_name="c", subcore_axis_name="s")


def _load_slab(src2d, idx, base, n_full):
    pltpu.sync_copy(src2d.at[pl.ds(base, n_full)], idx)


def _make_degree_kernel():
    @functools.partial(
        pl.kernel,
        out_type=jax.ShapeDtypeStruct((NC, NROWS, DEGW), jnp.float32),
        mesh=_sc_mesh(),
        scratch_types=[
            pltpu.VMEM((NCHW, CH), jnp.int32),
            pltpu.VMEM((CH, DEGW), jnp.float32),
            pltpu.VMEM_SHARED((NROWS, DEGW), jnp.float32),
        ] + [pltpu.SemaphoreType.DMA] * 2,
    )
    def deg_kernel(dst2d, ones, zeros, out, idx_d, onesv, acc, *sems):
        c = lax.axis_index("c")
        s = lax.axis_index("s")
        w = c * NS + s
        _load_slab(dst2d, idx_d, NCHW * w, NCHW)
        pltpu.sync_copy(ones, onesv)
        pltpu.sync_copy(zeros, acc.at[pl.ds(s * STRIPE, STRIPE)])
        plsc.subcore_barrier()

        def body(j, carry):
            for p in (0, 1):
                @pl.when(j % 2 == p)
                def _():
                    @pl.when(j >= 2)
                    def _():
                        pltpu.make_async_copy(
                            onesv, acc.at[idx_d.at[0]], sems[p]).wait()
                    pltpu.async_copy(onesv, acc.at[idx_d.at[j]], sems[p],
                                     add=True)
            return carry

        lax.fori_loop(0, NCHW, body, 0)
        for sem in sems:
            pltpu.make_async_copy(onesv, acc.at[idx_d.at[0]], sem).wait()
        plsc.subcore_barrier()
        pltpu.sync_copy(acc.at[pl.ds(s * STRIPE, STRIPE)],
                        out.at[c, pl.ds(s * STRIPE, STRIPE)])

    return deg_kernel


def _pipelined_chunk_loop(table, idx_s, idx_d, rows, acc, sems, n_chunks):
    sg = [sems[0], sems[1]]
    ss = [sems[2], sems[3]]
    pltpu.async_copy(table.at[idx_s.at[0]], rows.at[0], sg[0])
    pltpu.async_copy(table.at[idx_s.at[1]], rows.at[1], sg[1])

    def body(j, carry):
        for p in (0, 1, 2, 3):
            @pl.when(j % 4 == p)
            def _():
                h = p % 2
                pltpu.make_async_copy(
                    table.at[idx_s.at[0]], rows.at[p], sg[h]).wait()

                @pl.when(j + 2 < n_chunks)
                def _():
                    @pl.when(j >= 2)
                    def _():
                        pltpu.make_async_copy(
                            rows.at[(p + 2) % 4], acc.at[idx_d.at[0]],
                            ss[h]).wait()
                    pltpu.async_copy(table.at[idx_s.at[j + 2]],
                                     rows.at[(p + 2) % 4], sg[h])
                pltpu.async_copy(rows.at[p], acc.at[idx_d.at[j]],
                                 ss[h], add=True)
        return carry

    lax.fori_loop(0, n_chunks, body, 0)
    for sem in ss:
        for _ in range(2):
            pltpu.make_async_copy(rows.at[0], acc.at[idx_d.at[0]],
                                  sem).wait()


def _make_colsplit_kernel():
    @functools.partial(
        pl.kernel,
        out_type=jax.ShapeDtypeStruct((NROWS, NC, DHH), jnp.float32),
        mesh=_sc_mesh(),
        compiler_params=pltpu.CompilerParams(use_tc_tiling_on_sc=False),
        scratch_types=[
            pltpu.VMEM((NCHS, CH), jnp.int32),
            pltpu.VMEM((NCHS, CH), jnp.int32),
            pltpu.VMEM((4, CH, DHH), jnp.float32),
            pltpu.VMEM_SHARED((NROWS, DHH), jnp.float32),
        ] + [pltpu.SemaphoreType.DMA] * 4,
    )
    def gs_kernel(tflat, src2x, dst2d, zeros, out, idx_s, idx_d, rows, acc,
                  *sems):
        c = lax.axis_index("c")
        s = lax.axis_index("s")
        _load_slab(src2x.at[c], idx_s, NCHS * s, NCHS)
        _load_slab(dst2d, idx_d, NCHS * s, NCHS)
        pltpu.sync_copy(zeros, acc.at[pl.ds(s * STRIPE, STRIPE)])
        plsc.subcore_barrier()
        _pipelined_chunk_loop(tflat, idx_s, idx_d, rows, acc, sems, NCHS)
        plsc.subcore_barrier()
        pltpu.sync_copy(acc.at[pl.ds(s * STRIPE, STRIPE)],
                        out.at[pl.ds(s * STRIPE, STRIPE), c])

    return gs_kernel


def _make_edgesplit_kernel(D):
    @functools.partial(
        pl.kernel,
        out_type=jax.ShapeDtypeStruct((NROWS, NC, D), jnp.float32),
        mesh=_sc_mesh(),
        compiler_params=pltpu.CompilerParams(use_tc_tiling_on_sc=False),
        scratch_types=[
            pltpu.VMEM((NCHW, CH), jnp.int32),
            pltpu.VMEM((NCHW, CH), jnp.int32),
            pltpu.VMEM((4, CH, D), jnp.float32),
            pltpu.VMEM_SHARED((NROWS, D), jnp.float32),
        ] + [pltpu.SemaphoreType.DMA] * 4,
    )
    def gs_kernel(table, src2d, dst2d, zeros, out, idx_s, idx_d, rows, acc,
                  *sems):
        c = lax.axis_index("c")
        s = lax.axis_index("s")
        w = c * NS + s
        base = NCHW * w
        _load_slab(src2d, idx_s, base, NCHW)
        _load_slab(dst2d, idx_d, base, NCHW)
        pltpu.sync_copy(zeros, acc.at[pl.ds(s * STRIPE, STRIPE)])
        plsc.subcore_barrier()
        _pipelined_chunk_loop(table, idx_s, idx_d, rows, acc, sems, NCHW)
        plsc.subcore_barrier()
        pltpu.sync_copy(acc.at[pl.ds(s * STRIPE, STRIPE)],
                        out.at[pl.ds(s * STRIPE, STRIPE), c])

    return gs_kernel


def _dinv_block(deg_ref):
    deg = deg_ref[0, :, 0:1] + deg_ref[1, :, 0:1]
    return lax.rsqrt(deg)


def _tc1_body(deg_ref, x_ref, w_ref, h_ref, hp_ref):
    dinv = _dinv_block(deg_ref)
    h = jnp.dot(x_ref[...], w_ref[...], preferred_element_type=jnp.float32)
    h_ref[...] = h
    hp_ref[...] = h * dinv


def _tc2_body(deg_ref, s_ref, h1_ref, b_ref, w_ref, h2_ref, hp2_ref):
    dinv = _dinv_block(deg_ref)
    a = dinv * s_ref[...] + (dinv * dinv) * h1_ref[...] + b_ref[...]
    a = jnp.maximum(a, 0.0)
    h2 = jnp.dot(a, w_ref[...], preferred_element_type=jnp.float32)
    h2_ref[...] = h2
    hp2_ref[...] = h2 * dinv


def _tc3_body(deg_ref, s_ref, h2_ref, b_ref, o_ref):
    dinv = _dinv_block(deg_ref)
    sagg = s_ref[:, :DOUT] + s_ref[:, DOUT:]
    z = dinv * sagg + (dinv * dinv) * h2_ref[...] + b_ref[...]
    m = jnp.max(z, axis=1, keepdims=True)
    ez = jnp.exp(z - m)
    lse = jnp.log(jnp.sum(ez, axis=1, keepdims=True)) + m
    o_ref[...] = z - lse


def _deg_spec():
    return pl.BlockSpec((NC, RB, DEGW), lambda i: (0, i, 0))


def _full_spec(r, c):
    return pl.BlockSpec((r, c), lambda i: (0, 0))


def _row_spec(D):
    return pl.BlockSpec((RB, D), lambda i: (i, 0))


def kernel(x, edge_index, W1, b1, W2, b2):
    ei = edge_index.astype(jnp.int32)
    npad = NEP - NE
    pad = jnp.arange(npad, dtype=jnp.int32)
    src_pad = jnp.concatenate([ei[0], pad % N])
    dst_pad = jnp.concatenate([ei[1], N + pad % (NROWS - N)])
    src2d = src_pad.reshape(EROWS, CH)
    src2x = (src_pad * 2 + jnp.array([[0], [1]], jnp.int32)
             ).reshape(NC, EROWS, CH)
    dst2d = dst_pad.reshape(EROWS, CH)

    ones = jnp.ones((CH, DEGW), jnp.float32)
    zeros_deg = jnp.zeros((STRIPE, DEGW), jnp.float32)
    zeros_h = jnp.zeros((STRIPE, DHH), jnp.float32)
    zeros_o = jnp.zeros((STRIPE, DOUT), jnp.float32)

    degp = _make_degree_kernel()(dst2d, ones, zeros_deg)

    grid = (N // RB,)

    h1, hp1 = pl.pallas_call(
        _tc1_body,
        grid=grid,
        in_specs=[_deg_spec(), _row_spec(DIN), _full_spec(DIN, DH)],
        out_specs=[_row_spec(DH), _row_spec(DH)],
        out_shape=[jax.ShapeDtypeStruct((N, DH), jnp.float32)] * 2,
    )(degp, x, W1)

    s1i = _make_colsplit_kernel()(hp1.reshape(2 * N, DHH), src2x, dst2d,
                                  zeros_h)
    s1 = s1i.reshape(NROWS, DH)

    h2, hp2 = pl.pallas_call(
        _tc2_body,
        grid=grid,
        in_specs=[_deg_spec(), _row_spec(DH), _row_spec(DH),
                  _full_spec(1, DH), _full_spec(DH, DOUT)],
        out_specs=[_row_spec(DOUT), _row_spec(DOUT)],
        out_shape=[jax.ShapeDtypeStruct((N, DOUT), jnp.float32)] * 2,
    )(degp, s1, h1, b1.reshape(1, DH), W2)

    s2i = _make_edgesplit_kernel(DOUT)(hp2, src2d, dst2d, zeros_o)
    s2 = s2i.reshape(NROWS, 2 * DOUT)

    out = pl.pallas_call(
        _tc3_body,
        grid=grid,
        in_specs=[_deg_spec(), _row_spec(2 * DOUT), _row_spec(DOUT),
                  _full_spec(1, DOUT)],
        out_specs=_row_spec(DOUT),
        out_shape=jax.ShapeDtypeStruct((N, DOUT), jnp.float32),
    )(degp, s2, h2, b2.reshape(1, DOUT))

    return out


# speedup vs baseline: 1.2674x; 1.2674x over previous
import functools

import jax
import jax.numpy as jnp
from jax import lax
from jax.experimental import pallas as pl
from jax.experimental.pallas import tpu as pltpu
from jax.experimental.pallas import tpu_sc as plsc

N = 10000
NE = 320000
DIN, DH, DOUT = 128, 128, 64
DHH = DH // 2
NC, NS = 2, 16
NW = NC * NS
CH = 128
NCHW = 80
NCHS = 160
EROWS = NW * NCHW
NEP = EROWS * CH
NROWS = 10112
STRIPE = NROWS // NS
DEGW = 8
RB = 1000


def _sc_mesh():
    return plsc.VectorSubcoreMesh(core_axis_name="c", subcore_axis_name="s")


def _load_slab(src2d, idx, base, n_full):
    pltpu.sync_copy(src2d.at[pl.ds(base, n_full)], idx)


def _make_degree_kernel():
    @functools.partial(
        pl.kernel,
        out_type=jax.ShapeDtypeStruct((NC, NROWS, DEGW), jnp.float32),
        mesh=_sc_mesh(),
        scratch_types=[
            pltpu.VMEM((NCHW, CH), jnp.int32),
            pltpu.VMEM((CH, DEGW), jnp.float32),
            pltpu.VMEM_SHARED((NROWS, DEGW), jnp.float32),
        ] + [pltpu.SemaphoreType.DMA] * 2,
    )
    def deg_kernel(dst2d, ones, zeros, out, idx_d, onesv, acc, *sems):
        c = lax.axis_index("c")
        s = lax.axis_index("s")
        w = c * NS + s
        _load_slab(dst2d, idx_d, NCHW * w, NCHW)
        pltpu.sync_copy(ones, onesv)
        pltpu.sync_copy(zeros, acc.at[pl.ds(s * STRIPE, STRIPE)])
        plsc.subcore_barrier()

        def body(j, carry):
            for p in (0, 1):
                @pl.when(j % 2 == p)
                def _():
                    @pl.when(j >= 2)
                    def _():
                        pltpu.make_async_copy(
                            onesv, acc.at[idx_d.at[0]], sems[p]).wait()
                    pltpu.async_copy(onesv, acc.at[idx_d.at[j]], sems[p],
                                     add=True)
            return carry

        lax.fori_loop(0, NCHW, body, 0)
        for sem in sems:
            pltpu.make_async_copy(onesv, acc.at[idx_d.at[0]], sem).wait()
        plsc.subcore_barrier()
        pltpu.sync_copy(acc.at[pl.ds(s * STRIPE, STRIPE)],
                        out.at[c, pl.ds(s * STRIPE, STRIPE)])

    return deg_kernel


def _pipelined_chunk_loop(table, idx_s, idx_d, rows, acc, sems, n_chunks):
    sg = [sems[0], sems[1]]
    ss = [sems[2], sems[3]]
    pltpu.async_copy(table.at[idx_s.at[0]], rows.at[0], sg[0])
    pltpu.async_copy(table.at[idx_s.at[1]], rows.at[1], sg[1])

    def body(j, carry):
        for p in (0, 1, 2, 3):
            @pl.when(j % 4 == p)
            def _():
                h = p % 2
                pltpu.make_async_copy(
                    table.at[idx_s.at[0]], rows.at[p], sg[h]).wait()

                @pl.when(j + 2 < n_chunks)
                def _():
                    @pl.when(j >= 2)
                    def _():
                        pltpu.make_async_copy(
                            rows.at[(p + 2) % 4], acc.at[idx_d.at[0]],
                            ss[h]).wait()
                    pltpu.async_copy(table.at[idx_s.at[j + 2]],
                                     rows.at[(p + 2) % 4], sg[h])
                pltpu.async_copy(rows.at[p], acc.at[idx_d.at[j]],
                                 ss[h], add=True)
        return carry

    lax.fori_loop(0, n_chunks, body, 0)
    for sem in ss:
        for _ in range(2):
            pltpu.make_async_copy(rows.at[0], acc.at[idx_d.at[0]],
                                  sem).wait()


def _make_colsplit_kernel():
    @functools.partial(
        pl.kernel,
        out_type=jax.ShapeDtypeStruct((NROWS, DH), jnp.float32),
        mesh=_sc_mesh(),
        compiler_params=pltpu.CompilerParams(use_tc_tiling_on_sc=False),
        scratch_types=[
            pltpu.VMEM((NCHS, CH), jnp.int32),
            pltpu.VMEM((NCHS, CH), jnp.int32),
            pltpu.VMEM((4, CH, DHH), jnp.float32),
            pltpu.VMEM_SHARED((NROWS, DHH), jnp.float32),
        ] + [pltpu.SemaphoreType.DMA] * 4,
    )
    def gs_kernel(tflat, src2x, dst2d, zeros, out, idx_s, idx_d, rows, acc,
                  *sems):
        c = lax.axis_index("c")
        s = lax.axis_index("s")
        _load_slab(src2x.at[c], idx_s, NCHS * s, NCHS)
        _load_slab(dst2d, idx_d, NCHS * s, NCHS)
        pltpu.sync_copy(zeros, acc.at[pl.ds(s * STRIPE, STRIPE)])
        plsc.subcore_barrier()
        _pipelined_chunk_loop(tflat, idx_s, idx_d, rows, acc, sems, NCHS)
        plsc.subcore_barrier()
        pltpu.sync_copy(acc.at[pl.ds(s * STRIPE, STRIPE)],
                        out.at[pl.ds(s * STRIPE, STRIPE),
                               pl.ds(c * DHH, DHH)])

    return gs_kernel


def _make_edgesplit_kernel(D):
    @functools.partial(
        pl.kernel,
        out_type=jax.ShapeDtypeStruct((NROWS, NC * D), jnp.float32),
        mesh=_sc_mesh(),
        compiler_params=pltpu.CompilerParams(use_tc_tiling_on_sc=False),
        scratch_types=[
            pltpu.VMEM((NCHW, CH), jnp.int32),
            pltpu.VMEM((NCHW, CH), jnp.int32),
            pltpu.VMEM((4, CH, D), jnp.float32),
            pltpu.VMEM_SHARED((NROWS, D), jnp.float32),
        ] + [pltpu.SemaphoreType.DMA] * 4,
    )
    def gs_kernel(table, src2d, dst2d, zeros, out, idx_s, idx_d, rows, acc,
                  *sems):
        c = lax.axis_index("c")
        s = lax.axis_index("s")
        w = c * NS + s
        base = NCHW * w
        _load_slab(src2d, idx_s, base, NCHW)
        _load_slab(dst2d, idx_d, base, NCHW)
        pltpu.sync_copy(zeros, acc.at[pl.ds(s * STRIPE, STRIPE)])
        plsc.subcore_barrier()
        _pipelined_chunk_loop(table, idx_s, idx_d, rows, acc, sems, NCHW)
        plsc.subcore_barrier()
        pltpu.sync_copy(acc.at[pl.ds(s * STRIPE, STRIPE)],
                        out.at[pl.ds(s * STRIPE, STRIPE),
                               pl.ds(c * D, D)])

    return gs_kernel


def _dinv_block(deg_ref):
    deg = deg_ref[0, :, 0:1] + deg_ref[1, :, 0:1]
    return lax.rsqrt(deg)


def _tc1_body(deg_ref, x_ref, w_ref, h_ref, hp_ref):
    dinv = _dinv_block(deg_ref)
    h = jnp.dot(x_ref[...], w_ref[...], preferred_element_type=jnp.float32)
    h_ref[...] = h
    hp_ref[...] = h * dinv


def _tc2_body(deg_ref, s_ref, h1_ref, b_ref, w_ref, h2_ref, hp2_ref):
    dinv = _dinv_block(deg_ref)
    a = dinv * s_ref[...] + (dinv * dinv) * h1_ref[...] + b_ref[...]
    a = jnp.maximum(a, 0.0)
    h2 = jnp.dot(a, w_ref[...], preferred_element_type=jnp.float32)
    h2_ref[...] = h2
    hp2_ref[...] = h2 * dinv


def _tc3_body(deg_ref, s_ref, h2_ref, b_ref, o_ref):
    dinv = _dinv_block(deg_ref)
    sagg = s_ref[:, :DOUT] + s_ref[:, DOUT:]
    z = dinv * sagg + (dinv * dinv) * h2_ref[...] + b_ref[...]
    m = jnp.max(z, axis=1, keepdims=True)
    ez = jnp.exp(z - m)
    lse = jnp.log(jnp.sum(ez, axis=1, keepdims=True)) + m
    o_ref[...] = z - lse


def _deg_spec():
    return pl.BlockSpec((NC, RB, DEGW), lambda i: (0, i, 0))


def _full_spec(r, c):
    return pl.BlockSpec((r, c), lambda i: (0, 0))


def _row_spec(D):
    return pl.BlockSpec((RB, D), lambda i: (i, 0))


def kernel(x, edge_index, W1, b1, W2, b2):
    ei = edge_index.astype(jnp.int32)
    npad = NEP - NE
    pad = jnp.arange(npad, dtype=jnp.int32)
    src_pad = jnp.concatenate([ei[0], pad % N])
    dst_pad = jnp.concatenate([ei[1], N + pad % (NROWS - N)])
    src2d = src_pad.reshape(EROWS, CH)
    src2x = (src_pad * 2 + jnp.array([[0], [1]], jnp.int32)
             ).reshape(NC, EROWS, CH)
    dst2d = dst_pad.reshape(EROWS, CH)

    ones = jnp.ones((CH, DEGW), jnp.float32)
    zeros_deg = jnp.zeros((STRIPE, DEGW), jnp.float32)
    zeros_h = jnp.zeros((STRIPE, DHH), jnp.float32)
    zeros_o = jnp.zeros((STRIPE, DOUT), jnp.float32)

    degp = _make_degree_kernel()(dst2d, ones, zeros_deg)

    grid = (N // RB,)

    h1, hp1 = pl.pallas_call(
        _tc1_body,
        grid=grid,
        in_specs=[_deg_spec(), _row_spec(DIN), _full_spec(DIN, DH)],
        out_specs=[_row_spec(DH), _row_spec(DH)],
        out_shape=[jax.ShapeDtypeStruct((N, DH), jnp.float32)] * 2,
    )(degp, x, W1)

    s1 = _make_colsplit_kernel()(hp1.reshape(2 * N, DHH), src2x, dst2d,
                                 zeros_h)

    h2, hp2 = pl.pallas_call(
        _tc2_body,
        grid=grid,
        in_specs=[_deg_spec(), _row_spec(DH), _row_spec(DH),
                  _full_spec(1, DH), _full_spec(DH, DOUT)],
        out_specs=[_row_spec(DOUT), _row_spec(DOUT)],
        out_shape=[jax.ShapeDtypeStruct((N, DOUT), jnp.float32)] * 2,
    )(degp, s1, h1, b1.reshape(1, DH), W2)

    s2 = _make_edgesplit_kernel(DOUT)(hp2, src2d, dst2d, zeros_o)

    out = pl.pallas_call(
        _tc3_body,
        grid=grid,
        in_specs=[_deg_spec(), _row_spec(2 * DOUT), _row_spec(DOUT),
                  _full_spec(1, DOUT)],
        out_specs=_row_spec(DOUT),
        out_shape=jax.ShapeDtypeStruct((N, DOUT), jnp.float32),
    )(degp, s2, h2, b2.reshape(1, DOUT))

    return out


# speedup vs baseline: 1.2826x; 1.0119x over previous
import functools

import jax
import jax.numpy as jnp
from jax import lax
from jax.experimental import pallas as pl
from jax.experimental.pallas import tpu as pltpu
from jax.experimental.pallas import tpu_sc as plsc

N = 10000
NE = 320000
DIN, DH, DOUT = 128, 128, 64
DHH = DH // 2
NC, NS = 2, 16
NW = NC * NS
CH = 128
NCHW = 80
NCHS = 160
EROWSR = NE // CH
TR_W = EROWSR - NCHW * (NW - 1)
TR_S = EROWSR - NCHS * (NS - 1)
PADR = 64
NROWS = 10112
STRIPE = NROWS // NS
DEGW = 8
RB = 2000


def _sc_mesh():
    return plsc.VectorSubcoreMesh(core_axis_name="c", subcore_axis_name="s")


def _load_slab_w(real, pad, idx, w):
    @pl.when(w < NW - 1)
    def _():
        pltpu.sync_copy(real.at[pl.ds(NCHW * w, NCHW)], idx)

    @pl.when(w == NW - 1)
    def _():
        pltpu.sync_copy(pad, idx.at[pl.ds(NCHW - PADR, PADR)])
        pltpu.sync_copy(real.at[pl.ds(NCHW * (NW - 1), TR_W)],
                        idx.at[pl.ds(0, TR_W)])


def _load_slab_s(real, pad, idx, sub):
    @pl.when(sub < NS - 1)
    def _():
        pltpu.sync_copy(real.at[pl.ds(NCHS * sub, NCHS)], idx)

    @pl.when(sub == NS - 1)
    def _():
        pltpu.sync_copy(pad, idx.at[pl.ds(NCHS - PADR, PADR)])
        pltpu.sync_copy(real.at[pl.ds(NCHS * (NS - 1), TR_S)],
                        idx.at[pl.ds(0, TR_S)])


def _make_degree_kernel():
    @functools.partial(
        pl.kernel,
        out_type=jax.ShapeDtypeStruct((NC, NROWS, DEGW), jnp.float32),
        mesh=_sc_mesh(),
        scratch_types=[
            pltpu.VMEM((NCHW, CH), jnp.int32),
            pltpu.VMEM((CH, DEGW), jnp.float32),
            pltpu.VMEM_SHARED((NROWS, DEGW), jnp.float32),
        ] + [pltpu.SemaphoreType.DMA] * 2,
    )
    def deg_kernel(dstr, dstpad, ones, zeros, out, idx_d, onesv, acc,
                   *sems):
        c = lax.axis_index("c")
        s = lax.axis_index("s")
        w = c * NS + s
        _load_slab_w(dstr, dstpad, idx_d, w)
        pltpu.sync_copy(ones, onesv)
        pltpu.sync_copy(zeros, acc.at[pl.ds(s * STRIPE, STRIPE)])
        plsc.subcore_barrier()

        def body(j, carry):
            for p in (0, 1):
                @pl.when(j % 2 == p)
                def _():
                    @pl.when(j >= 2)
                    def _():
                        pltpu.make_async_copy(
                            onesv, acc.at[idx_d.at[0]], sems[p]).wait()
                    pltpu.async_copy(onesv, acc.at[idx_d.at[j]], sems[p],
                                     add=True)
            return carry

        lax.fori_loop(0, NCHW, body, 0)
        for sem in sems:
            pltpu.make_async_copy(onesv, acc.at[idx_d.at[0]], sem).wait()
        plsc.subcore_barrier()
        pltpu.sync_copy(acc.at[pl.ds(s * STRIPE, STRIPE)],
                        out.at[c, pl.ds(s * STRIPE, STRIPE)])

    return deg_kernel


def _pipelined_chunk_loop(table, idx_s, idx_d, rows, acc, sems, n_chunks):
    sg = [sems[0], sems[1]]
    ss = [sems[2], sems[3]]
    pltpu.async_copy(table.at[idx_s.at[0]], rows.at[0], sg[0])
    pltpu.async_copy(table.at[idx_s.at[1]], rows.at[1], sg[1])

    def body(j, carry):
        for p in (0, 1, 2, 3):
            @pl.when(j % 4 == p)
            def _():
                h = p % 2
                pltpu.make_async_copy(
                    table.at[idx_s.at[0]], rows.at[p], sg[h]).wait()

                @pl.when(j + 2 < n_chunks)
                def _():
                    @pl.when(j >= 2)
                    def _():
                        pltpu.make_async_copy(
                            rows.at[(p + 2) % 4], acc.at[idx_d.at[0]],
                            ss[h]).wait()
                    pltpu.async_copy(table.at[idx_s.at[j + 2]],
                                     rows.at[(p + 2) % 4], sg[h])
                pltpu.async_copy(rows.at[p], acc.at[idx_d.at[j]],
                                 ss[h], add=True)
        return carry

    lax.fori_loop(0, n_chunks, body, 0)
    for sem in ss:
        for _ in range(2):
            pltpu.make_async_copy(rows.at[0], acc.at[idx_d.at[0]],
                                  sem).wait()


def _make_colsplit_kernel():
    @functools.partial(
        pl.kernel,
        out_type=jax.ShapeDtypeStruct((NROWS, DH), jnp.float32),
        mesh=_sc_mesh(),
        compiler_params=pltpu.CompilerParams(use_tc_tiling_on_sc=False),
        scratch_types=[
            pltpu.VMEM((NCHS, CH), jnp.int32),
            pltpu.VMEM((NCHS, CH), jnp.int32),
            pltpu.VMEM((4, CH, DHH), jnp.float32),
            pltpu.VMEM_SHARED((NROWS, DHH), jnp.float32),
        ] + [pltpu.SemaphoreType.DMA] * 4,
    )
    def gs_kernel(tflat, src2x, src2xpad, dstr, dstpad, zeros, out,
                  idx_s, idx_d, rows, acc, *sems):
        c = lax.axis_index("c")
        s = lax.axis_index("s")
        _load_slab_s(src2x.at[c], src2xpad.at[c], idx_s, s)
        _load_slab_s(dstr, dstpad, idx_d, s)
        pltpu.sync_copy(zeros, acc.at[pl.ds(s * STRIPE, STRIPE)])
        plsc.subcore_barrier()
        _pipelined_chunk_loop(tflat, idx_s, idx_d, rows, acc, sems, NCHS)
        plsc.subcore_barrier()
        pltpu.sync_copy(acc.at[pl.ds(s * STRIPE, STRIPE)],
                        out.at[pl.ds(s * STRIPE, STRIPE),
                               pl.ds(c * DHH, DHH)])

    return gs_kernel


def _make_edgesplit_kernel(D):
    @functools.partial(
        pl.kernel,
        out_type=jax.ShapeDtypeStruct((NROWS, NC * D), jnp.float32),
        mesh=_sc_mesh(),
        compiler_params=pltpu.CompilerParams(use_tc_tiling_on_sc=False),
        scratch_types=[
            pltpu.VMEM((NCHW, CH), jnp.int32),
            pltpu.VMEM((NCHW, CH), jnp.int32),
            pltpu.VMEM((4, CH, D), jnp.float32),
            pltpu.VMEM_SHARED((NROWS, D), jnp.float32),
        ] + [pltpu.SemaphoreType.DMA] * 4,
    )
    def gs_kernel(table, srcr, srcpad, dstr, dstpad, zeros, out,
                  idx_s, idx_d, rows, acc, *sems):
        c = lax.axis_index("c")
        s = lax.axis_index("s")
        w = c * NS + s
        _load_slab_w(srcr, srcpad, idx_s, w)
        _load_slab_w(dstr, dstpad, idx_d, w)
        pltpu.sync_copy(zeros, acc.at[pl.ds(s * STRIPE, STRIPE)])
        plsc.subcore_barrier()
        _pipelined_chunk_loop(table, idx_s, idx_d, rows, acc, sems, NCHW)
        plsc.subcore_barrier()
        pltpu.sync_copy(acc.at[pl.ds(s * STRIPE, STRIPE)],
                        out.at[pl.ds(s * STRIPE, STRIPE),
                               pl.ds(c * D, D)])

    return gs_kernel


def _dinv_block(deg_ref):
    deg = deg_ref[0, :, 0:1] + deg_ref[1, :, 0:1]
    return lax.rsqrt(deg)


def _tc1_body(deg_ref, x_ref, w_ref, h_ref, hp_ref):
    dinv = _dinv_block(deg_ref)
    h = jnp.dot(x_ref[...], w_ref[...], preferred_element_type=jnp.float32)
    h_ref[...] = h
    hp_ref[...] = h * dinv


def _tc2_body(deg_ref, s_ref, h1_ref, b_ref, w_ref, h2_ref, hp2_ref):
    dinv = _dinv_block(deg_ref)
    a = dinv * s_ref[...] + (dinv * dinv) * h1_ref[...] + b_ref[...]
    a = jnp.maximum(a, 0.0)
    h2 = jnp.dot(a, w_ref[...], preferred_element_type=jnp.float32)
    h2_ref[...] = h2
    hp2_ref[...] = h2 * dinv


def _tc3_body(deg_ref, s_ref, h2_ref, b_ref, o_ref):
    dinv = _dinv_block(deg_ref)
    sagg = s_ref[:, :DOUT] + s_ref[:, DOUT:]
    z = dinv * sagg + (dinv * dinv) * h2_ref[...] + b_ref[...]
    m = jnp.max(z, axis=1, keepdims=True)
    ez = jnp.exp(z - m)
    lse = jnp.log(jnp.sum(ez, axis=1, keepdims=True)) + m
    o_ref[...] = z - lse


def _deg_spec():
    return pl.BlockSpec((NC, RB, DEGW), lambda i: (0, i, 0))


def _full_spec(r, c):
    return pl.BlockSpec((r, c), lambda i: (0, 0))


def _row_spec(D):
    return pl.BlockSpec((RB, D), lambda i: (i, 0))


def kernel(x, edge_index, W1, b1, W2, b2):
    ei = edge_index.astype(jnp.int32)
    padv = jnp.arange(PADR * CH, dtype=jnp.int32)
    dst_pad = (N + padv % (NROWS - N)).reshape(PADR, CH)
    src_pad = (padv % N).reshape(PADR, CH)
    src2x_pad = ((padv % N) * 2
                 + jnp.array([[[0]], [[1]]], jnp.int32)
                 ).reshape(NC, PADR, CH)
    dst_r = ei[1].reshape(EROWSR, CH)
    src_r = ei[0].reshape(EROWSR, CH)

    ones = jnp.ones((CH, DEGW), jnp.float32)
    zeros_deg = jnp.zeros((STRIPE, DEGW), jnp.float32)
    zeros_h = jnp.zeros((STRIPE, DHH), jnp.float32)
    zeros_o = jnp.zeros((STRIPE, DOUT), jnp.float32)

    degp = _make_degree_kernel()(dst_r, dst_pad, ones, zeros_deg)

    src2x_r = (ei[0] * 2 + jnp.array([[0], [1]], jnp.int32)
               ).reshape(NC, EROWSR, CH)

    grid = (N // RB,)

    h1, hp1 = pl.pallas_call(
        _tc1_body,
        grid=grid,
        in_specs=[_deg_spec(), _row_spec(DIN), _full_spec(DIN, DH)],
        out_specs=[_row_spec(DH), _row_spec(DH)],
        out_shape=[jax.ShapeDtypeStruct((N, DH), jnp.float32)] * 2,
    )(degp, x, W1)

    s1 = _make_colsplit_kernel()(hp1.reshape(2 * N, DHH), src2x_r,
                                 src2x_pad, dst_r, dst_pad, zeros_h)

    h2, hp2 = pl.pallas_call(
        _tc2_body,
        grid=grid,
        in_specs=[_deg_spec(), _row_spec(DH), _row_spec(DH),
                  _full_spec(1, DH), _full_spec(DH, DOUT)],
        out_specs=[_row_spec(DOUT), _row_spec(DOUT)],
        out_shape=[jax.ShapeDtypeStruct((N, DOUT), jnp.float32)] * 2,
    )(degp, s1, h1, b1.reshape(1, DH), W2)

    s2 = _make_edgesplit_kernel(DOUT)(hp2, src_r, src_pad, dst_r,
                                      dst_pad, zeros_o)

    out = pl.pallas_call(
        _tc3_body,
        grid=grid,
        in_specs=[_deg_spec(), _row_spec(2 * DOUT), _row_spec(DOUT),
                  _full_spec(1, DOUT)],
        out_specs=_row_spec(DOUT),
        out_shape=jax.ShapeDtypeStruct((N, DOUT), jnp.float32),
    )(degp, s2, h2, b2.reshape(1, DOUT))

    return out


# speedup vs baseline: 1.3602x; 1.0605x over previous
import functools

import jax
import jax.numpy as jnp
from jax import lax
from jax.experimental import pallas as pl
from jax.experimental.pallas import tpu as pltpu
from jax.experimental.pallas import tpu_sc as plsc

N = 10000
NE = 320000
DIN, DH, DOUT = 128, 128, 64
DHH = DH // 2
NC, NS = 2, 16
NW = NC * NS
CH = 128
NCHW = 80
NCHS = 160
EROWSR = NE // CH
TR_W = EROWSR - NCHW * (NW - 1)
TR_S = EROWSR - NCHS * (NS - 1)
PADR = 64
NROWS = 10112
STRIPE = NROWS // NS
DEGW = 8
RB = 2000


def _sc_mesh():
    return plsc.VectorSubcoreMesh(core_axis_name="c", subcore_axis_name="s")


def _load_slab_w(real, pad, idx, w):
    @pl.when(w < NW - 1)
    def _():
        pltpu.sync_copy(real.at[pl.ds(NCHW * w, NCHW)], idx)

    @pl.when(w == NW - 1)
    def _():
        pltpu.sync_copy(pad, idx.at[pl.ds(NCHW - PADR, PADR)])
        pltpu.sync_copy(real.at[pl.ds(NCHW * (NW - 1), TR_W)],
                        idx.at[pl.ds(0, TR_W)])


def _load_slab_s(real, pad, idx, sub):
    @pl.when(sub < NS - 1)
    def _():
        pltpu.sync_copy(real.at[pl.ds(NCHS * sub, NCHS)], idx)

    @pl.when(sub == NS - 1)
    def _():
        pltpu.sync_copy(pad, idx.at[pl.ds(NCHS - PADR, PADR)])
        pltpu.sync_copy(real.at[pl.ds(NCHS * (NS - 1), TR_S)],
                        idx.at[pl.ds(0, TR_S)])


def _make_degree_kernel():
    @functools.partial(
        pl.kernel,
        out_type=jax.ShapeDtypeStruct((NC, NROWS, DEGW), jnp.float32),
        mesh=_sc_mesh(),
        scratch_types=[
            pltpu.VMEM((NCHW, CH), jnp.int32),
            pltpu.VMEM((CH, DEGW), jnp.float32),
            pltpu.VMEM_SHARED((NROWS, DEGW), jnp.float32),
        ] + [pltpu.SemaphoreType.DMA] * 2,
    )
    def deg_kernel(dstr, dstpad, ones, zeros, out, idx_d, onesv, acc,
                   *sems):
        c = lax.axis_index("c")
        s = lax.axis_index("s")
        w = c * NS + s
        _load_slab_w(dstr, dstpad, idx_d, w)
        pltpu.sync_copy(ones, onesv)
        pltpu.sync_copy(zeros, acc.at[pl.ds(s * STRIPE, STRIPE)])
        plsc.subcore_barrier()

        def body(j, carry):
            for p in (0, 1):
                @pl.when(j % 2 == p)
                def _():
                    @pl.when(j >= 2)
                    def _():
                        pltpu.make_async_copy(
                            onesv, acc.at[idx_d.at[0]], sems[p]).wait()
                    pltpu.async_copy(onesv, acc.at[idx_d.at[j]], sems[p],
                                     add=True)
            return carry

        lax.fori_loop(0, NCHW, body, 0)
        for sem in sems:
            pltpu.make_async_copy(onesv, acc.at[idx_d.at[0]], sem).wait()
        plsc.subcore_barrier()
        pltpu.sync_copy(acc.at[pl.ds(s * STRIPE, STRIPE)],
                        out.at[c, pl.ds(s * STRIPE, STRIPE)])

    return deg_kernel


def _pipelined_chunk_loop(table, idx_s, idx_d, rows, acc, sems, n_chunks):
    sg = sems[:3]
    ss = sems[3:]
    for k in range(3):
        pltpu.async_copy(table.at[idx_s.at[k]], rows.at[k], sg[k])

    def body(j, carry):
        for p in range(6):
            @pl.when(j % 6 == p)
            def _():
                h = p % 3
                pltpu.make_async_copy(
                    table.at[idx_s.at[0]], rows.at[p], sg[h]).wait()

                @pl.when(j + 3 < n_chunks)
                def _():
                    @pl.when(j >= 3)
                    def _():
                        pltpu.make_async_copy(
                            rows.at[(p + 3) % 6], acc.at[idx_d.at[0]],
                            ss[h]).wait()
                    pltpu.async_copy(table.at[idx_s.at[j + 3]],
                                     rows.at[(p + 3) % 6], sg[h])
                pltpu.async_copy(rows.at[p], acc.at[idx_d.at[j]],
                                 ss[h], add=True)
        return carry

    lax.fori_loop(0, n_chunks, body, 0)
    for sem in ss:
        for _ in range(2):
            pltpu.make_async_copy(rows.at[0], acc.at[idx_d.at[0]],
                                  sem).wait()


def _make_colsplit_kernel():
    @functools.partial(
        pl.kernel,
        out_type=jax.ShapeDtypeStruct((NROWS, DH), jnp.float32),
        mesh=_sc_mesh(),
        compiler_params=pltpu.CompilerParams(use_tc_tiling_on_sc=False),
        scratch_types=[
            pltpu.VMEM((NCHS, CH), jnp.int32),
            pltpu.VMEM((NCHS, CH), jnp.int32),
            pltpu.VMEM((6, CH, DHH), jnp.float32),
            pltpu.VMEM_SHARED((NROWS, DHH), jnp.float32),
        ] + [pltpu.SemaphoreType.DMA] * 6,
    )
    def gs_kernel(tflat, src2x, src2xpad, dstr, dstpad, zeros, out,
                  idx_s, idx_d, rows, acc, *sems):
        c = lax.axis_index("c")
        s = lax.axis_index("s")
        _load_slab_s(src2x.at[c], src2xpad.at[c], idx_s, s)
        _load_slab_s(dstr, dstpad, idx_d, s)
        pltpu.sync_copy(zeros, acc.at[pl.ds(s * STRIPE, STRIPE)])
        plsc.subcore_barrier()
        _pipelined_chunk_loop(tflat, idx_s, idx_d, rows, acc, sems, NCHS)
        plsc.subcore_barrier()
        pltpu.sync_copy(acc.at[pl.ds(s * STRIPE, STRIPE)],
                        out.at[pl.ds(s * STRIPE, STRIPE),
                               pl.ds(c * DHH, DHH)])

    return gs_kernel


def _make_edgesplit_kernel(D):
    @functools.partial(
        pl.kernel,
        out_type=jax.ShapeDtypeStruct((NROWS, NC * D), jnp.float32),
        mesh=_sc_mesh(),
        compiler_params=pltpu.CompilerParams(use_tc_tiling_on_sc=False),
        scratch_types=[
            pltpu.VMEM((NCHW, CH), jnp.int32),
            pltpu.VMEM((NCHW, CH), jnp.int32),
            pltpu.VMEM((6, CH, D), jnp.float32),
            pltpu.VMEM_SHARED((NROWS, D), jnp.float32),
        ] + [pltpu.SemaphoreType.DMA] * 6,
    )
    def gs_kernel(table, srcr, srcpad, dstr, dstpad, zeros, out,
                  idx_s, idx_d, rows, acc, *sems):
        c = lax.axis_index("c")
        s = lax.axis_index("s")
        w = c * NS + s
        _load_slab_w(srcr, srcpad, idx_s, w)
        _load_slab_w(dstr, dstpad, idx_d, w)
        pltpu.sync_copy(zeros, acc.at[pl.ds(s * STRIPE, STRIPE)])
        plsc.subcore_barrier()
        _pipelined_chunk_loop(table, idx_s, idx_d, rows, acc, sems, NCHW)
        plsc.subcore_barrier()
        pltpu.sync_copy(acc.at[pl.ds(s * STRIPE, STRIPE)],
                        out.at[pl.ds(s * STRIPE, STRIPE),
                               pl.ds(c * D, D)])

    return gs_kernel


def _dinv_block(deg_ref):
    deg = deg_ref[0, :, 0:1] + deg_ref[1, :, 0:1]
    return lax.rsqrt(deg)


def _tc1_body(deg_ref, x_ref, w_ref, h_ref, hp_ref):
    dinv = _dinv_block(deg_ref)
    h = jnp.dot(x_ref[...], w_ref[...], preferred_element_type=jnp.float32)
    h_ref[...] = h
    hp_ref[...] = h * dinv


def _tc2_body(deg_ref, s_ref, h1_ref, b_ref, w_ref, h2_ref, hp2_ref):
    dinv = _dinv_block(deg_ref)
    a = dinv * s_ref[...] + (dinv * dinv) * h1_ref[...] + b_ref[...]
    a = jnp.maximum(a, 0.0)
    h2 = jnp.dot(a, w_ref[...], preferred_element_type=jnp.float32)
    h2_ref[...] = h2
    hp2_ref[...] = h2 * dinv


def _tc3_body(deg_ref, s_ref, h2_ref, b_ref, o_ref):
    dinv = _dinv_block(deg_ref)
    sagg = s_ref[:, :DOUT] + s_ref[:, DOUT:]
    z = dinv * sagg + (dinv * dinv) * h2_ref[...] + b_ref[...]
    m = jnp.max(z, axis=1, keepdims=True)
    ez = jnp.exp(z - m)
    lse = jnp.log(jnp.sum(ez, axis=1, keepdims=True)) + m
    o_ref[...] = z - lse


def _deg_spec():
    return pl.BlockSpec((NC, RB, DEGW), lambda i: (0, i, 0))


def _full_spec(r, c):
    return pl.BlockSpec((r, c), lambda i: (0, 0))


def _row_spec(D):
    return pl.BlockSpec((RB, D), lambda i: (i, 0))


def kernel(x, edge_index, W1, b1, W2, b2):
    ei = edge_index.astype(jnp.int32)
    padv = jnp.arange(PADR * CH, dtype=jnp.int32)
    dst_pad = (N + padv % (NROWS - N)).reshape(PADR, CH)
    src_pad = (padv % N).reshape(PADR, CH)
    src2x_pad = ((padv % N) * 2
                 + jnp.array([[[0]], [[1]]], jnp.int32)
                 ).reshape(NC, PADR, CH)
    dst_r = ei[1].reshape(EROWSR, CH)
    src_r = ei[0].reshape(EROWSR, CH)

    ones = jnp.ones((CH, DEGW), jnp.float32)
    zeros_deg = jnp.zeros((STRIPE, DEGW), jnp.float32)
    zeros_h = jnp.zeros((STRIPE, DHH), jnp.float32)
    zeros_o = jnp.zeros((STRIPE, DOUT), jnp.float32)

    degp = _make_degree_kernel()(dst_r, dst_pad, ones, zeros_deg)

    src2x_r = (ei[0] * 2 + jnp.array([[0], [1]], jnp.int32)
               ).reshape(NC, EROWSR, CH)

    grid = (N // RB,)

    h1, hp1 = pl.pallas_call(
        _tc1_body,
        grid=grid,
        in_specs=[_deg_spec(), _row_spec(DIN), _full_spec(DIN, DH)],
        out_specs=[_row_spec(DH), _row_spec(DH)],
        out_shape=[jax.ShapeDtypeStruct((N, DH), jnp.float32)] * 2,
    )(degp, x, W1)

    s1 = _make_colsplit_kernel()(hp1.reshape(2 * N, DHH), src2x_r,
                                 src2x_pad, dst_r, dst_pad, zeros_h)

    h2, hp2 = pl.pallas_call(
        _tc2_body,
        grid=grid,
        in_specs=[_deg_spec(), _row_spec(DH), _row_spec(DH),
                  _full_spec(1, DH), _full_spec(DH, DOUT)],
        out_specs=[_row_spec(DOUT), _row_spec(DOUT)],
        out_shape=[jax.ShapeDtypeStruct((N, DOUT), jnp.float32)] * 2,
    )(degp, s1, h1, b1.reshape(1, DH), W2)

    s2 = _make_edgesplit_kernel(DOUT)(hp2, src_r, src_pad, dst_r,
                                      dst_pad, zeros_o)

    out = pl.pallas_call(
        _tc3_body,
        grid=grid,
        in_specs=[_deg_spec(), _row_spec(2 * DOUT), _row_spec(DOUT),
                  _full_spec(1, DOUT)],
        out_specs=_row_spec(DOUT),
        out_shape=jax.ShapeDtypeStruct((N, DOUT), jnp.float32),
    )(degp, s2, h2, b2.reshape(1, DOUT))

    return out
